# Initial kernel scaffold; baseline (speedup 1.0000x reference)
import jax
import jax.numpy as jnp
from jax.experimental import pallas as pl


def kernel(x, edge_index, edge_weight, weight, bias):
    raise NotImplementedError("write your pallas kernel here")


# speedup vs baseline: 8.7936x; 8.7936x over previous
import functools

import jax
import jax.numpy as jnp
from jax import lax
from jax.experimental import pallas as pl
from jax.experimental.pallas import tpu as pltpu
from jax.experimental.pallas import tpu_sc as plsc

N = 10000
E = 320000
C = 128
NC = 2
NS = 16
L = 16

CH = 128
NCH = 79
EPW = CH * NCH
EPAD = EPW * NC * NS
DEG_NCH = 2 * NCH
DEG_EPT = CH * DEG_NCH

ROWS_PT = N // NS
DSL = 632


def _rsqrt_newton(x):
    i = lax.bitcast_convert_type(x, jnp.int32)
    i = jnp.int32(0x5F3759DF) - lax.shift_right_arithmetic(i, jnp.int32(1))
    y = lax.bitcast_convert_type(i, jnp.float32)
    for _ in range(3):
        y = y * (1.5 - 0.5 * x * y * y)
    return y


def _sc_spmm(xm, rowp, colp, ewp):
    mesh = plsc.VectorSubcoreMesh(
        core_axis_name="c", subcore_axis_name="s", num_cores=NC,
        num_subcores=NS)

    @functools.partial(
        pl.kernel,
        out_type=jax.ShapeDtypeStruct((NC, N, C), jnp.float32),
        mesh=mesh,
        compiler_params=pltpu.CompilerParams(needs_layout_passes=False),
        scratch_types=[
            pltpu.VMEM_SHARED((N,), jnp.float32),
            pltpu.VMEM_SHARED((N, C), jnp.float32),
            pltpu.VMEM((N,), jnp.float32),
            pltpu.VMEM((CH, C), jnp.float32),
            pltpu.VMEM((CH,), jnp.int32),
            pltpu.VMEM((CH,), jnp.int32),
            pltpu.VMEM((CH,), jnp.float32),
            pltpu.VMEM((CH,), jnp.float32),
            pltpu.VMEM((640,), jnp.float32),
            pltpu.SemaphoreType.DMA,
        ],
    )
    def k(x_hbm, row_hbm, col_hbm, ew_hbm, m_out,
          deg_sp, m_sp, dinv_v, rows_v, rowc_v, colc_v, ewc_v, c_v,
          zz_v, sem):
        cid = lax.axis_index("c")
        sid = lax.axis_index("s")
        wid = sid * NC + cid
        zero16 = jnp.zeros((L,), jnp.float32)
        iota16 = lax.iota(jnp.int32, L)

        for t in range(640 // L):
            zz_v[pl.ds(t * L, L)] = zero16

        def _zrow(i, _):
            for j in range(C // L):
                rows_v[i, pl.ds(j * L, L)] = zero16
            return 0
        lax.fori_loop(0, CH, _zrow, 0)

        base_r = sid * DSL

        @pl.when(sid < NS - 1)
        def _zero_main():
            pltpu.sync_copy(zz_v.at[pl.ds(0, DSL)],
                            deg_sp.at[pl.ds(base_r, DSL)])
            for t in range(4):
                pltpu.sync_copy(rows_v, m_sp.at[pl.ds(base_r + t * CH, CH)])
            pltpu.sync_copy(rows_v.at[pl.ds(0, DSL - 4 * CH)],
                            m_sp.at[pl.ds(base_r + 4 * CH, DSL - 4 * CH)])

        @pl.when(sid == NS - 1)
        def _zero_last():
            rem = N - (NS - 1) * DSL
            pltpu.sync_copy(zz_v.at[pl.ds(0, rem)],
                            deg_sp.at[pl.ds(base_r, rem)])
            for t in range(4):
                pltpu.sync_copy(rows_v, m_sp.at[pl.ds(base_r + t * CH, CH)])
            pltpu.sync_copy(rows_v.at[pl.ds(0, rem - 4 * CH)],
                            m_sp.at[pl.ds(base_r + 4 * CH, rem - 4 * CH)])

        plsc.subcore_barrier()

        dbase = sid * DEG_EPT

        def _deg(g, _):
            e0 = dbase + g * CH
            pltpu.sync_copy(row_hbm.at[pl.ds(e0, CH)], rowc_v)
            pltpu.sync_copy(ew_hbm.at[pl.ds(e0, CH)], ewc_v)
            pltpu.sync_copy(ewc_v, deg_sp.at[rowc_v], add=True)
            return 0
        lax.fori_loop(0, DEG_NCH, _deg, 0)
        plsc.subcore_barrier()

        pltpu.sync_copy(deg_sp, dinv_v)

        def _dinv(t, _):
            d = dinv_v[pl.ds(t * L, L)]
            y = _rsqrt_newton(d)
            dinv_v[pl.ds(t * L, L)] = jnp.where(d > 0.0, y, 0.0)
            return 0
        lax.fori_loop(0, N // L, _dinv, 0)

        ebase = wid * EPW

        def _edge_chunk(g, _):
            e0 = ebase + g * CH
            pltpu.sync_copy(row_hbm.at[pl.ds(e0, CH)], rowc_v)
            pltpu.sync_copy(col_hbm.at[pl.ds(e0, CH)], colc_v)
            pltpu.sync_copy(ew_hbm.at[pl.ds(e0, CH)], ewc_v)
            for j in range(CH // L):
                sl = pl.ds(j * L, L)
                dr = plsc.load_gather(dinv_v, [rowc_v[sl]])
                dc = plsc.load_gather(dinv_v, [colc_v[sl]])
                c_v[sl] = jnp.float32(-2.0) * dr * ewc_v[sl] * dc
            pltpu.async_copy(x_hbm.at[colc_v], rows_v, sem).wait()

            def _scale(i, _):
                cv = plsc.load_gather(c_v, [jnp.full((L,), i, jnp.int32)])
                for j in range(C // L):
                    sl = pl.ds(j * L, L)
                    rows_v[i, sl] = rows_v[i, sl] * cv
                return 0
            lax.fori_loop(0, CH, _scale, 0)
            pltpu.sync_copy(rows_v, m_sp.at[rowc_v], add=True)
            return 0
        lax.fori_loop(0, NCH, _edge_chunk, 0)
        plsc.subcore_barrier()

        @pl.when(sid < NS - 1)
        def _out_main():
            pltpu.sync_copy(m_sp.at[pl.ds(base_r, DSL)],
                            m_out.at[cid, pl.ds(base_r, DSL)])

        @pl.when(sid == NS - 1)
        def _out_last():
            rem = N - (NS - 1) * DSL
            pltpu.sync_copy(m_sp.at[pl.ds(base_r, rem)],
                            m_out.at[cid, pl.ds(base_r, rem)])

    return k(xm, rowp, colp, ewp)


def _tc_combine_kernel(x_ref, m_ref, w_ref, b_ref, o_ref):
    wb = w_ref[1] + 2.0 * w_ref[2] + w_ref[3]
    wa = w_ref[0] - w_ref[2] - 0.1 * wb
    m = m_ref[0] + m_ref[1]
    o_ref[...] = (
        jnp.dot(x_ref[...], wa, preferred_element_type=jnp.float32)
        + jnp.dot(m, wb, preferred_element_type=jnp.float32)
        + b_ref[...])


def _tc_combine(xm, mparts, weight, bias):
    R = 2000
    return pl.pallas_call(
        _tc_combine_kernel,
        grid=(N // R,),
        in_specs=[
            pl.BlockSpec((R, C), lambda i: (i, 0)),
            pl.BlockSpec((NC, R, C), lambda i: (0, i, 0)),
            pl.BlockSpec((4, C, C), lambda i: (0, 0, 0)),
            pl.BlockSpec((1, C), lambda i: (0, 0)),
        ],
        out_specs=pl.BlockSpec((R, C), lambda i: (i, 0)),
        out_shape=jax.ShapeDtypeStruct((N, C), jnp.float32),
    )(xm, mparts, weight, bias)


def kernel(x, edge_index, edge_weight, weight, bias):
    xm = x.reshape(N, C)
    row = edge_index[0].astype(jnp.int32)
    col = edge_index[1].astype(jnp.int32)
    ew = edge_weight.astype(jnp.float32)
    pad = EPAD - E
    rowp = jnp.pad(row, (0, pad))
    colp = jnp.pad(col, (0, pad))
    ewp = jnp.pad(ew, (0, pad))
    mparts = _sc_spmm(xm, rowp, colp, ewp)
    out = _tc_combine(xm, mparts, weight, bias.reshape(1, C))
    return out.reshape(1, N, C)


# speedup vs baseline: 24.8125x; 2.8217x over previous
import functools

import jax
import jax.numpy as jnp
import numpy as np
from jax import lax
from jax.experimental import pallas as pl
from jax.experimental.pallas import tpu as pltpu
from jax.experimental.pallas import tpu_sc as plsc

N = 10000
E = 320000
C = 128
NC = 2
NS = 16
L = 16

CH = 128
NCH = 80
NBLK = 16
NB = NCH // NBLK
EPW = CH * NCH
NCHT = NCH * NC * NS
EPAD = CH * NCHT
DEG_NCH = 2 * NCH
DEG_NB = DEG_NCH // NBLK

ROWS_PT = N // NS
DSL = 632


def _rsqrt_newton(x):
    i = lax.bitcast_convert_type(x, jnp.int32)
    i = jnp.int32(0x5F3759DF) - lax.shift_right_arithmetic(i, jnp.int32(1))
    y = lax.bitcast_convert_type(i, jnp.float32)
    for _ in range(3):
        y = y * (1.5 - 0.5 * x * y * y)
    return y


def _sc_spmm(xm, rowp, colp, ewp):
    mesh = plsc.VectorSubcoreMesh(
        core_axis_name="c", subcore_axis_name="s", num_cores=NC,
        num_subcores=NS)

    @functools.partial(
        pl.kernel,
        out_type=jax.ShapeDtypeStruct((NC, N, C), jnp.float32),
        mesh=mesh,
        compiler_params=pltpu.CompilerParams(needs_layout_passes=False),
        scratch_types=[
            pltpu.VMEM_SHARED((N,), jnp.float32),
            pltpu.VMEM_SHARED((N, C), jnp.float32),
            pltpu.VMEM((N,), jnp.float32),
            pltpu.VMEM((NBLK, CH), jnp.int32),
            pltpu.VMEM((NBLK, CH), jnp.int32),
            pltpu.VMEM((NBLK, CH), jnp.float32),
            pltpu.VMEM((CH, C), jnp.float32),
            pltpu.VMEM((CH, C), jnp.float32),
            pltpu.VMEM((CH,), jnp.float32),
            pltpu.VMEM((640,), jnp.float32),
            pltpu.SemaphoreType.DMA,
            pltpu.SemaphoreType.DMA,
            pltpu.SemaphoreType.DMA,
            pltpu.SemaphoreType.DMA,
        ],
    )
    def k(x_hbm, row_hbm, col_hbm, ew_hbm, m_out,
          deg_sp, m_sp, dinv_v, rowm_v, colm_v, cm_v, buf0_v, buf1_v,
          cc_v, zz_v, semA0, semA1, semS0, semS1):
        cid = lax.axis_index("c")
        sid = lax.axis_index("s")
        wid = sid * NC + cid
        zero16 = jnp.zeros((L,), jnp.float32)

        for t in range(640 // L):
            zz_v[pl.ds(t * L, L)] = zero16

        def _zrow(i, _):
            for j in range(C // L):
                buf0_v[i, pl.ds(j * L, L)] = zero16
            return 0
        lax.fori_loop(0, CH, _zrow, 0)

        base_r = sid * DSL

        @pl.when(sid < NS - 1)
        def _zero_main():
            pltpu.sync_copy(zz_v.at[pl.ds(0, DSL)],
                            deg_sp.at[pl.ds(base_r, DSL)])
            for t in range(4):
                pltpu.sync_copy(buf0_v, m_sp.at[pl.ds(base_r + t * CH, CH)])
            pltpu.sync_copy(buf0_v.at[pl.ds(0, DSL - 4 * CH)],
                            m_sp.at[pl.ds(base_r + 4 * CH, DSL - 4 * CH)])

        @pl.when(sid == NS - 1)
        def _zero_last():
            rem = N - (NS - 1) * DSL
            pltpu.sync_copy(zz_v.at[pl.ds(0, rem)],
                            deg_sp.at[pl.ds(base_r, rem)])
            for t in range(4):
                pltpu.sync_copy(buf0_v, m_sp.at[pl.ds(base_r + t * CH, CH)])
            pltpu.sync_copy(buf0_v.at[pl.ds(0, rem - 4 * CH)],
                            m_sp.at[pl.ds(base_r + 4 * CH, rem - 4 * CH)])

        plsc.subcore_barrier()

        dchunk0 = sid * DEG_NCH

        def _deg_blk(b, _):
            bc0 = dchunk0 + b * NBLK
            pltpu.sync_copy(row_hbm.at[pl.ds(bc0, NBLK)], rowm_v)
            pltpu.sync_copy(ew_hbm.at[pl.ds(bc0, NBLK)], cm_v)

            def _deg(g, _):
                pltpu.sync_copy(cm_v.at[g], deg_sp.at[rowm_v.at[g]],
                                add=True)
                return 0
            lax.fori_loop(0, NBLK, _deg, 0)
            return 0
        lax.fori_loop(0, DEG_NB, _deg_blk, 0)
        plsc.subcore_barrier()

        pltpu.sync_copy(deg_sp, dinv_v)

        def _dinv(t, _):
            d = dinv_v[pl.ds(t * L, L)]
            y = _rsqrt_newton(d)
            dinv_v[pl.ds(t * L, L)] = jnp.where(d > 0.0, y, 0.0)
            return 0
        lax.fori_loop(0, N // L, _dinv, 0)

        def _scale(buf, g):
            for j in range(CH // L):
                sl = pl.ds(j * L, L)
                cc_v[sl] = cm_v[g, sl]

            def _rows(i8, _):
                for u in range(8):
                    i = i8 * 8 + u
                    cv = plsc.load_gather(
                        cc_v, [jnp.full((L,), i, jnp.int32)])
                    for j in range(C // L):
                        sl = pl.ds(j * L, L)
                        buf[i, sl] = buf[i, sl] * cv
                return 0
            lax.fori_loop(0, CH // 8, _rows, 0)

        wc0 = wid * NCH

        def _blk(b, _):
            bc0 = wc0 + b * NBLK
            pltpu.sync_copy(row_hbm.at[pl.ds(bc0, NBLK)], rowm_v)
            pltpu.sync_copy(col_hbm.at[pl.ds(bc0, NBLK)], colm_v)
            pltpu.sync_copy(ew_hbm.at[pl.ds(bc0, NBLK)], cm_v)

            def _coef(g, _):
                for j in range(CH // L):
                    sl = pl.ds(j * L, L)
                    dr = plsc.load_gather(dinv_v, [rowm_v[g, sl]])
                    dc = plsc.load_gather(dinv_v, [colm_v[g, sl]])
                    cm_v[g, sl] = jnp.float32(-2.0) * dr * cm_v[g, sl] * dc
                return 0
            lax.fori_loop(0, NBLK, _coef, 0)

            h0 = pltpu.async_copy(x_hbm.at[colm_v.at[0]], buf0_v, semA0)
            h0.wait()

            def _pair(h, _):
                c0 = 2 * h
                c1 = 2 * h + 1
                a1 = pltpu.async_copy(x_hbm.at[colm_v.at[c1]], buf1_v,
                                      semA1)
                _scale(buf0_v, c0)
                s0 = pltpu.async_copy(buf0_v, m_sp.at[rowm_v.at[c0]],
                                      semS0, add=True)
                a1.wait()
                _scale(buf1_v, c1)
                s0.wait()
                cn = jnp.minimum(c1 + 1, NBLK - 1)
                a0 = pltpu.async_copy(x_hbm.at[colm_v.at[cn]], buf0_v,
                                      semA0)
                s1 = pltpu.async_copy(buf1_v, m_sp.at[rowm_v.at[c1]],
                                      semS1, add=True)
                a0.wait()
                s1.wait()
                return 0
            lax.fori_loop(0, NBLK // 2, _pair, 0)
            return 0
        lax.fori_loop(0, NB, _blk, 0)
        plsc.subcore_barrier()

        @pl.when(sid < NS - 1)
        def _out_main():
            pltpu.sync_copy(m_sp.at[pl.ds(base_r, DSL)],
                            m_out.at[cid, pl.ds(base_r, DSL)])

        @pl.when(sid == NS - 1)
        def _out_last():
            rem = N - (NS - 1) * DSL
            pltpu.sync_copy(m_sp.at[pl.ds(base_r, rem)],
                            m_out.at[cid, pl.ds(base_r, rem)])

    return k(xm, rowp, colp, ewp)


def _tc_combine_kernel(x_ref, m_ref, w_ref, b_ref, o_ref):
    wb = w_ref[1] + 2.0 * w_ref[2] + w_ref[3]
    wa = w_ref[0] - w_ref[2] - 0.1 * wb
    m = m_ref[0] + m_ref[1]
    o_ref[...] = (
        jnp.dot(x_ref[...], wa, preferred_element_type=jnp.float32)
        + jnp.dot(m, wb, preferred_element_type=jnp.float32)
        + b_ref[...])


def _tc_combine(xm, mparts, weight, bias):
    R = 2000
    return pl.pallas_call(
        _tc_combine_kernel,
        grid=(N // R,),
        in_specs=[
            pl.BlockSpec((R, C), lambda i: (i, 0)),
            pl.BlockSpec((NC, R, C), lambda i: (0, i, 0)),
            pl.BlockSpec((4, C, C), lambda i: (0, 0, 0)),
            pl.BlockSpec((1, C), lambda i: (0, 0)),
        ],
        out_specs=pl.BlockSpec((R, C), lambda i: (i, 0)),
        out_shape=jax.ShapeDtypeStruct((N, C), jnp.float32),
    )(xm, mparts, weight, bias)


_PAD_IDX = np.arange(EPAD - E, dtype=np.int32) % N


def kernel(x, edge_index, edge_weight, weight, bias):
    xm = x.reshape(N, C)
    row = edge_index[0].astype(jnp.int32)
    col = edge_index[1].astype(jnp.int32)
    ew = edge_weight.astype(jnp.float32)
    rowp = jnp.concatenate([row, _PAD_IDX]).reshape(NCHT, CH)
    colp = jnp.concatenate([col, _PAD_IDX]).reshape(NCHT, CH)
    ewp = jnp.concatenate(
        [ew, jnp.zeros((EPAD - E,), jnp.float32)]).reshape(NCHT, CH)
    mparts = _sc_spmm(xm, rowp, colp, ewp)
    out = _tc_combine(xm, mparts, weight, bias.reshape(1, C))
    return out.reshape(1, N, C)


# speedup vs baseline: 26.1950x; 1.0557x over previous
import functools

import jax
import jax.numpy as jnp
import numpy as np
from jax import lax
from jax.experimental import pallas as pl
from jax.experimental.pallas import tpu as pltpu
from jax.experimental.pallas import tpu_sc as plsc

N = 10000
E = 320000
C = 128
NC = 2
NS = 16
L = 16

CH = 128
NCH = 80
NBLK = 16
NB = NCH // NBLK
EPW = CH * NCH
NCHT = NCH * NC * NS
EPAD = CH * NCHT
DEG_NCH = 2 * NCH
DEG_NB = DEG_NCH // NBLK

ROWS_PT = N // NS
DSL = 632


def _rsqrt_newton(x):
    i = lax.bitcast_convert_type(x, jnp.int32)
    i = jnp.int32(0x5F3759DF) - lax.shift_right_arithmetic(i, jnp.int32(1))
    y = lax.bitcast_convert_type(i, jnp.float32)
    for _ in range(3):
        y = y * (1.5 - 0.5 * x * y * y)
    return y


def _sc_spmm(xm, rowp, colp, ewp):
    mesh = plsc.VectorSubcoreMesh(
        core_axis_name="c", subcore_axis_name="s", num_cores=NC,
        num_subcores=NS)

    @functools.partial(
        pl.kernel,
        out_type=jax.ShapeDtypeStruct((NC, N, C), jnp.float32),
        mesh=mesh,
        compiler_params=pltpu.CompilerParams(needs_layout_passes=False),
        scratch_types=[
            pltpu.VMEM_SHARED((N,), jnp.float32),
            pltpu.VMEM_SHARED((N, C), jnp.float32),
            pltpu.VMEM((N,), jnp.float32),
            pltpu.VMEM((NBLK, CH), jnp.int32),
            pltpu.VMEM((NBLK, CH), jnp.int32),
            pltpu.VMEM((NBLK, CH), jnp.float32),
            pltpu.VMEM((CH, C), jnp.float32),
            pltpu.VMEM((CH, C), jnp.float32),
            pltpu.VMEM((CH,), jnp.float32),
            pltpu.VMEM((640,), jnp.float32),
            pltpu.SemaphoreType.DMA,
            pltpu.SemaphoreType.DMA,
            pltpu.SemaphoreType.DMA,
            pltpu.SemaphoreType.DMA,
        ],
    )
    def k(x_hbm, row_hbm, col_hbm, ew_hbm, m_out,
          deg_sp, m_sp, dinv_v, rowm_v, colm_v, cm_v, buf0_v, buf1_v,
          cc_v, zz_v, semA0, semA1, semS0, semS1):
        cid = lax.axis_index("c")
        sid = lax.axis_index("s")
        wid = sid * NC + cid
        zero16 = jnp.zeros((L,), jnp.float32)

        for t in range(640 // L):
            zz_v[pl.ds(t * L, L)] = zero16

        def _zrow(i, _):
            for j in range(C // L):
                buf0_v[i, pl.ds(j * L, L)] = zero16
            return 0
        lax.fori_loop(0, CH, _zrow, 0)

        base_r = sid * DSL

        @pl.when(sid < NS - 1)
        def _zero_main():
            pltpu.sync_copy(zz_v.at[pl.ds(0, DSL)],
                            deg_sp.at[pl.ds(base_r, DSL)])
            for t in range(4):
                pltpu.sync_copy(buf0_v, m_sp.at[pl.ds(base_r + t * CH, CH)])
            pltpu.sync_copy(buf0_v.at[pl.ds(0, DSL - 4 * CH)],
                            m_sp.at[pl.ds(base_r + 4 * CH, DSL - 4 * CH)])

        @pl.when(sid == NS - 1)
        def _zero_last():
            rem = N - (NS - 1) * DSL
            pltpu.sync_copy(zz_v.at[pl.ds(0, rem)],
                            deg_sp.at[pl.ds(base_r, rem)])
            for t in range(4):
                pltpu.sync_copy(buf0_v, m_sp.at[pl.ds(base_r + t * CH, CH)])
            pltpu.sync_copy(buf0_v.at[pl.ds(0, rem - 4 * CH)],
                            m_sp.at[pl.ds(base_r + 4 * CH, rem - 4 * CH)])

        plsc.subcore_barrier()

        HB = NBLK // 2
        dchunk0 = sid * DEG_NCH
        NHB = DEG_NCH // HB

        def _dload(b, half, sem):
            bc0 = dchunk0 + b * HB
            hr = pltpu.async_copy(row_hbm.at[pl.ds(bc0, HB)],
                                  rowm_v.at[pl.ds(half * HB, HB)], sem)
            he = pltpu.async_copy(ew_hbm.at[pl.ds(bc0, HB)],
                                  cm_v.at[pl.ds(half * HB, HB)], sem)
            return hr, he

        def _dscatter(half, sem):
            hs = []
            for g in range(HB):
                hs.append(pltpu.async_copy(
                    cm_v.at[half * HB + g],
                    deg_sp.at[rowm_v.at[half * HB + g]], sem, add=True))
            for h in hs:
                h.wait()

        pr, pe = _dload(0, 0, semA0)
        pr.wait()
        pe.wait()

        def _degpair(p, _):
            b1 = 2 * p + 1
            r1, e1 = _dload(b1, 1, semA1)
            _dscatter(0, semS0)
            r1.wait()
            e1.wait()
            bn = jnp.minimum(2 * p + 2, NHB - 1)
            r0, e0 = _dload(bn, 0, semA0)
            _dscatter(1, semS1)
            r0.wait()
            e0.wait()
            return 0
        lax.fori_loop(0, NHB // 2, _degpair, 0)
        plsc.subcore_barrier()

        pltpu.sync_copy(deg_sp, dinv_v)

        def _dinv(t, _):
            d = dinv_v[pl.ds(t * L, L)]
            y = _rsqrt_newton(d)
            dinv_v[pl.ds(t * L, L)] = jnp.where(d > 0.0, y, 0.0)
            return 0
        lax.fori_loop(0, N // L, _dinv, 0)

        def _scale(buf, g):
            for j in range(CH // L):
                sl = pl.ds(j * L, L)
                cc_v[sl] = cm_v[g, sl]

            def _rows(i8, _):
                for u in range(8):
                    i = i8 * 8 + u
                    cv = plsc.load_gather(
                        cc_v, [jnp.full((L,), i, jnp.int32)])
                    for j in range(C // L):
                        sl = pl.ds(j * L, L)
                        buf[i, sl] = buf[i, sl] * cv
                return 0
            lax.fori_loop(0, CH // 8, _rows, 0)

        wc0 = wid * NCH

        def _blk(b, _):
            bc0 = wc0 + b * NBLK
            pltpu.sync_copy(row_hbm.at[pl.ds(bc0, NBLK)], rowm_v)
            pltpu.sync_copy(col_hbm.at[pl.ds(bc0, NBLK)], colm_v)
            pltpu.sync_copy(ew_hbm.at[pl.ds(bc0, NBLK)], cm_v)

            def _coef(g, _):
                for j in range(CH // L):
                    sl = pl.ds(j * L, L)
                    dr = plsc.load_gather(dinv_v, [rowm_v[g, sl]])
                    dc = plsc.load_gather(dinv_v, [colm_v[g, sl]])
                    cm_v[g, sl] = jnp.float32(-2.0) * dr * cm_v[g, sl] * dc
                return 0
            lax.fori_loop(0, NBLK, _coef, 0)

            h0 = pltpu.async_copy(x_hbm.at[colm_v.at[0]], buf0_v, semA0)
            h0.wait()

            def _pair(h, _):
                c0 = 2 * h
                c1 = 2 * h + 1
                a1 = pltpu.async_copy(x_hbm.at[colm_v.at[c1]], buf1_v,
                                      semA1)
                _scale(buf0_v, c0)
                s0 = pltpu.async_copy(buf0_v, m_sp.at[rowm_v.at[c0]],
                                      semS0, add=True)
                a1.wait()
                _scale(buf1_v, c1)
                s0.wait()
                cn = jnp.minimum(c1 + 1, NBLK - 1)
                a0 = pltpu.async_copy(x_hbm.at[colm_v.at[cn]], buf0_v,
                                      semA0)
                s1 = pltpu.async_copy(buf1_v, m_sp.at[rowm_v.at[c1]],
                                      semS1, add=True)
                a0.wait()
                s1.wait()
                return 0
            lax.fori_loop(0, NBLK // 2, _pair, 0)
            return 0
        lax.fori_loop(0, NB, _blk, 0)
        plsc.subcore_barrier()

        @pl.when(sid < NS - 1)
        def _out_main():
            pltpu.sync_copy(m_sp.at[pl.ds(base_r, DSL)],
                            m_out.at[cid, pl.ds(base_r, DSL)])

        @pl.when(sid == NS - 1)
        def _out_last():
            rem = N - (NS - 1) * DSL
            pltpu.sync_copy(m_sp.at[pl.ds(base_r, rem)],
                            m_out.at[cid, pl.ds(base_r, rem)])

    return k(xm, rowp, colp, ewp)


def _tc_combine_kernel(x_ref, m_ref, w_ref, b_ref, o_ref):
    wb = w_ref[1] + 2.0 * w_ref[2] + w_ref[3]
    wa = w_ref[0] - w_ref[2] - 0.1 * wb
    m = m_ref[0] + m_ref[1]
    o_ref[...] = (
        jnp.dot(x_ref[...], wa, preferred_element_type=jnp.float32)
        + jnp.dot(m, wb, preferred_element_type=jnp.float32)
        + b_ref[...])


def _tc_combine(xm, mparts, weight, bias):
    R = 2000
    return pl.pallas_call(
        _tc_combine_kernel,
        grid=(N // R,),
        in_specs=[
            pl.BlockSpec((R, C), lambda i: (i, 0)),
            pl.BlockSpec((NC, R, C), lambda i: (0, i, 0)),
            pl.BlockSpec((4, C, C), lambda i: (0, 0, 0)),
            pl.BlockSpec((1, C), lambda i: (0, 0)),
        ],
        out_specs=pl.BlockSpec((R, C), lambda i: (i, 0)),
        out_shape=jax.ShapeDtypeStruct((N, C), jnp.float32),
    )(xm, mparts, weight, bias)


_PAD_IDX = np.arange(EPAD - E, dtype=np.int32) % N


def kernel(x, edge_index, edge_weight, weight, bias):
    xm = x.reshape(N, C)
    row = edge_index[0].astype(jnp.int32)
    col = edge_index[1].astype(jnp.int32)
    ew = edge_weight.astype(jnp.float32)
    rowp = jnp.concatenate([row, _PAD_IDX]).reshape(NCHT, CH)
    colp = jnp.concatenate([col, _PAD_IDX]).reshape(NCHT, CH)
    ewp = jnp.concatenate(
        [ew, jnp.zeros((EPAD - E,), jnp.float32)]).reshape(NCHT, CH)
    mparts = _sc_spmm(xm, rowp, colp, ewp)
    out = _tc_combine(xm, mparts, weight, bias.reshape(1, C))
    return out.reshape(1, N, C)
